# Initial kernel scaffold; baseline (speedup 1.0000x reference)
import jax
import jax.numpy as jnp
from jax.experimental import pallas as pl


def kernel(hidden_states, gate_w, expert_w, expert_b):
    raise NotImplementedError("write your pallas kernel here")


# speedup vs baseline: 1.2803x; 1.2803x over previous
import dataclasses
import functools

import jax
import jax.numpy as jnp
from jax.experimental import pallas as pl
from jax.experimental.pallas import tpu as pltpu
from jax.experimental.pallas import tpu_sc as plsc

E = 8
TOP_K = 2
D = 2048
OUT = 2048
BM = 512
BN = 1024
NEG = -1e30


def _a_body(h_ref, g_ref, logits_ref, hbf_ref, lsum_ref):
    i = pl.program_id(0)
    h = h_ref[...]
    hb = h.astype(jnp.bfloat16)
    hbf_ref[...] = hb
    lg = jnp.dot(hb, g_ref[...], preferred_element_type=jnp.float32)
    logits_ref[...] = lg
    cs = jnp.sum(lg, axis=0, keepdims=True)
    cs16 = jnp.concatenate([cs, jnp.full((1, E), NEG, jnp.float32)], axis=1)

    @pl.when(i == 0)
    def _():
        lsum_ref[...] = cs16

    @pl.when(i != 0)
    def _():
        lsum_ref[...] = lsum_ref[...] + cs16


def _stage_a(h2d, gwt_bf16, n_tokens):
    m_tiles = n_tokens // BM
    return pl.pallas_call(
        _a_body,
        grid=(m_tiles,),
        in_specs=[
            pl.BlockSpec((BM, D), lambda i: (i, 0)),
            pl.BlockSpec((D, E), lambda i: (0, 0)),
        ],
        out_specs=[
            pl.BlockSpec((BM, E), lambda i: (i, 0)),
            pl.BlockSpec((BM, D), lambda i: (i, 0)),
            pl.BlockSpec((1, 2 * E), lambda i: (0, 0)),
        ],
        out_shape=[
            jax.ShapeDtypeStruct((n_tokens, E), jnp.float32),
            jax.ShapeDtypeStruct((n_tokens, D), jnp.bfloat16),
            jax.ShapeDtypeStruct((1, 2 * E), jnp.float32),
        ],
        compiler_params=pltpu.CompilerParams(
            dimension_semantics=("arbitrary",),
        ),
    )(h2d, gwt_bf16)


def _b_body(lsum_hbm, sel_hbm, lsum_vmem, sel_vmem, sem):
    core = jax.lax.axis_index("c")
    sub = jax.lax.axis_index("s")

    @pl.when(jnp.logical_and(core == 0, sub == 0))
    def _():
        pltpu.async_copy(lsum_hbm, lsum_vmem, sem).wait()
        keys = lsum_vmem[...]
        idx = jax.lax.iota(jnp.int32, 16)
        _, sv = plsc.sort_key_val(keys, idx, descending=True)
        sel_vmem[...] = sv
        pltpu.async_copy(sel_vmem, sel_hbm, sem).wait()


def _stage_b_topk(lsum16):
    cp = pltpu.CompilerParams()
    if "needs_layout_passes" in pltpu.CompilerParams.__dataclass_fields__:
        cp = dataclasses.replace(cp, needs_layout_passes=False)
    mesh = plsc.VectorSubcoreMesh(core_axis_name="c", subcore_axis_name="s")
    fn = pl.kernel(
        _b_body,
        out_type=jax.ShapeDtypeStruct((16,), jnp.int32),
        mesh=mesh,
        scratch_types=[
            pltpu.VMEM((16,), jnp.float32),
            pltpu.VMEM((16,), jnp.int32),
            pltpu.SemaphoreType.DMA,
        ],
        compiler_params=cp,
    )
    return fn(lsum16)


def _c_body(sel_ref, h_ref, lg_ref, w0_ref, w1_ref, b_ref, o_ref, wb0, wb1):
    m = pl.program_id(1)

    @pl.when(m == 0)
    def _():
        wb0[...] = w0_ref[0].astype(jnp.bfloat16)
        wb1[...] = w1_ref[0].astype(jnp.bfloat16)

    h = h_ref[...]
    acc0 = jnp.dot(h, wb0[...], preferred_element_type=jnp.float32)
    acc1 = jnp.dot(h, wb1[...], preferred_element_type=jnp.float32)

    sel0 = sel_ref[0]
    sel1 = sel_ref[1]
    lg = lg_ref[...]
    lane = jax.lax.broadcasted_iota(jnp.int32, lg.shape, 1)
    l0 = jnp.sum(jnp.where(lane == sel0, lg, 0.0), axis=1, keepdims=True)
    l1 = jnp.sum(jnp.where(lane == sel1, lg, 0.0), axis=1, keepdims=True)
    mx = jnp.maximum(l0, l1)
    e0 = jnp.exp(l0 - mx)
    e1 = jnp.exp(l1 - mx)
    inv = 1.0 / (e0 + e1)
    w0 = e0 * inv
    w1 = e1 * inv

    b = b_ref[...]
    row = jax.lax.broadcasted_iota(jnp.int32, b.shape, 0)
    b0 = jnp.sum(jnp.where(row == sel0, b, 0.0), axis=0, keepdims=True)
    b1 = jnp.sum(jnp.where(row == sel1, b, 0.0), axis=0, keepdims=True)

    o_ref[...] = w0 * (acc0 + b0) + w1 * (acc1 + b1)


def _stage_c(sel16, hbf, logits, expert_w, expert_b, n_tokens):
    m_tiles = n_tokens // BM
    n_tiles = OUT // BN
    grid_spec = pltpu.PrefetchScalarGridSpec(
        num_scalar_prefetch=1,
        grid=(n_tiles, m_tiles),
        in_specs=[
            pl.BlockSpec((BM, D), lambda n, m, sel: (m, 0)),
            pl.BlockSpec((BM, E), lambda n, m, sel: (m, 0)),
            pl.BlockSpec((1, D, BN), lambda n, m, sel: (sel[0], 0, n)),
            pl.BlockSpec((1, D, BN), lambda n, m, sel: (sel[1], 0, n)),
            pl.BlockSpec((E, BN), lambda n, m, sel: (0, n)),
        ],
        out_specs=pl.BlockSpec((BM, BN), lambda n, m, sel: (m, n)),
        scratch_shapes=[
            pltpu.VMEM((D, BN), jnp.bfloat16),
            pltpu.VMEM((D, BN), jnp.bfloat16),
        ],
    )
    return pl.pallas_call(
        _c_body,
        grid_spec=grid_spec,
        out_shape=jax.ShapeDtypeStruct((n_tokens, OUT), jnp.float32),
        compiler_params=pltpu.CompilerParams(
            dimension_semantics=("arbitrary", "arbitrary"),
        ),
    )(sel16, hbf, logits, expert_w, expert_w, expert_b)


@functools.partial(jax.jit, static_argnames=())
def kernel(hidden_states, gate_w, expert_w, expert_b):
    b, s, d = hidden_states.shape
    n_tokens = b * s
    h2d = hidden_states.reshape(n_tokens, d)
    gwt_bf16 = gate_w.T.astype(jnp.bfloat16)

    logits, hbf, lsum16 = _stage_a(h2d, gwt_bf16, n_tokens)
    sel16 = _stage_b_topk(lsum16.reshape(16))
    out2d = _stage_c(sel16, hbf, logits, expert_w, expert_b, n_tokens)
    return out2d.reshape(b, s, OUT)
